# Initial kernel scaffold; baseline (speedup 1.0000x reference)
import jax
import jax.numpy as jnp
from jax.experimental import pallas as pl


def kernel(x, indices, weight, bias):
    raise NotImplementedError("write your pallas kernel here")


# speedup vs baseline: 2.0801x; 2.0801x over previous
import dataclasses
import functools

import jax
import jax.numpy as jnp
from jax import lax
from jax.experimental import pallas as pl
from jax.experimental.pallas import tpu as pltpu
from jax.experimental.pallas import tpu_sc as plsc

B = 1024
IN_FEATURES = 4096
OUT_FEATURES = 2048
K = 32
NC = 2
NS = 16
NW = NC * NS
RPW = OUT_FEATURES // NW
L = 16


def _splat(ref, idxs):
    return plsc.load_gather(ref, [jnp.full((L,), i, jnp.int32) for i in idxs])


def _compiler_params():
    cp = pltpu.CompilerParams()
    if "needs_layout_passes" in pltpu.CompilerParams.__dataclass_fields__:
        cp = dataclasses.replace(cp, needs_layout_passes=False)
    return cp


def _sc_sparse_linear(xT, idx, w, bias):
    mesh = plsc.VectorSubcoreMesh(core_axis_name="c", subcore_axis_name="s")

    @functools.partial(
        pl.kernel,
        out_type=jax.ShapeDtypeStruct((OUT_FEATURES, B), jnp.float32),
        mesh=mesh,
        compiler_params=_compiler_params(),
        scratch_types=[
            pltpu.VMEM((RPW, K), jnp.int32),
            pltpu.VMEM((RPW, K), jnp.float32),
            pltpu.VMEM((RPW,), jnp.float32),
            pltpu.VMEM((2, K, B), jnp.float32),
            pltpu.VMEM((B,), jnp.float32),
            pltpu.SemaphoreType.DMA,
            pltpu.SemaphoreType.DMA,
        ],
    )
    def body(xT_hbm, idx_hbm, w_hbm, bias_hbm, out_hbm,
             idx_v, w_v, bias_v, g_v, acc_v, sem0, sem1):
        wid = lax.axis_index("s") * NC + lax.axis_index("c")
        base = wid * RPW
        pltpu.sync_copy(idx_hbm.at[pl.ds(base, RPW)], idx_v)
        pltpu.sync_copy(w_hbm.at[pl.ds(base, RPW)], w_v)
        pltpu.sync_copy(bias_hbm.at[pl.ds(base, RPW)], bias_v)

        sems = (sem0, sem1)
        pltpu.async_copy(xT_hbm.at[idx_v.at[0]], g_v.at[0], sem0)

        @pl.loop(0, RPW, step=2)
        def _(r0):
            for p in range(2):
                r = r0 + p
                buf = g_v.at[p]
                pltpu.make_async_copy(xT_hbm.at[idx_v.at[r]], buf,
                                      sems[p]).wait()
                nxt = jnp.minimum(r + 1, RPW - 1)
                pltpu.async_copy(xT_hbm.at[idx_v.at[nxt]], g_v.at[1 - p],
                                 sems[1 - p])

                bvec = _splat(bias_v, (r,))
                wvecs = [_splat(w_v, (r, k)) for k in range(K)]

                @pl.loop(0, B, step=L)
                def _(c):
                    acc = bvec
                    for k in range(K):
                        acc = acc + wvecs[k] * buf[k, pl.ds(c, L)]
                    acc_v[pl.ds(c, L)] = acc

                pltpu.sync_copy(acc_v, out_hbm.at[base + r])

        pltpu.make_async_copy(xT_hbm.at[idx_v.at[RPW - 1]], g_v.at[0],
                              sem0).wait()

    return body(xT, idx, w, bias)


def kernel(x, indices, weight, bias):
    xT = x.T
    idx = indices.astype(jnp.int32)
    outT = _sc_sparse_linear(xT, idx, weight.astype(jnp.float32),
                             bias.astype(jnp.float32))
    return outT.T


# speedup vs baseline: 7.2371x; 3.4792x over previous
import dataclasses
import functools

import jax
import jax.numpy as jnp
from jax import lax
from jax.experimental import pallas as pl
from jax.experimental.pallas import tpu as pltpu
from jax.experimental.pallas import tpu_sc as plsc

B = 1024
IN_FEATURES = 4096
OUT_FEATURES = 2048
K = 32
NC = 2
NS = 16
NW = NC * NS
RPW = OUT_FEATURES // NW
L = 16

TO = 256


def _compiler_params():
    cp = pltpu.CompilerParams()
    if "needs_layout_passes" in pltpu.CompilerParams.__dataclass_fields__:
        cp = dataclasses.replace(cp, needs_layout_passes=False)
    return cp


def _sc_scatter_weights(idx, w):
    mesh = plsc.VectorSubcoreMesh(core_axis_name="c", subcore_axis_name="s")

    @functools.partial(
        pl.kernel,
        out_type=jax.ShapeDtypeStruct((OUT_FEATURES, IN_FEATURES), jnp.float32),
        mesh=mesh,
        compiler_params=_compiler_params(),
        scratch_types=[
            pltpu.VMEM((RPW, K), jnp.int32),
            pltpu.VMEM((RPW, K), jnp.float32),
            pltpu.VMEM((IN_FEATURES,), jnp.float32),
            pltpu.VMEM((IN_FEATURES,), jnp.float32),
            pltpu.SemaphoreType.DMA,
            pltpu.SemaphoreType.DMA,
        ],
    )
    def body(idx_hbm, w_hbm, s_hbm, idx_v, w_v, rb0_v, rb1_v, sem0, sem1):
        rbufs = (rb0_v, rb1_v)
        wid = lax.axis_index("s") * NC + lax.axis_index("c")
        base = wid * RPW
        pltpu.sync_copy(idx_hbm.at[pl.ds(base, RPW)], idx_v)
        pltpu.sync_copy(w_hbm.at[pl.ds(base, RPW)], w_v)

        zeros = jnp.zeros((L,), jnp.float32)

        @pl.loop(0, IN_FEATURES, step=L)
        def _(c):
            rb0_v[pl.ds(c, L)] = zeros
            rb1_v[pl.ds(c, L)] = zeros

        def scatter_row(r, p):
            for h in range(K // L):
                iv = idx_v[r, pl.ds(h * L, L)]
                wv = w_v[r, pl.ds(h * L, L)]
                plsc.addupdate_scatter(rbufs[p], [iv], wv)

        def unscatter_row(r, p):
            for h in range(K // L):
                iv = idx_v[r, pl.ds(h * L, L)]
                plsc.store_scatter(rbufs[p], [iv], zeros)

        sems = (sem0, sem1)

        def start_dma(r, p):
            pltpu.async_copy(rbufs[p], s_hbm.at[base + r], sems[p])

        def wait_dma(r, p):
            pltpu.make_async_copy(rbufs[p], s_hbm.at[base + r],
                                  sems[p]).wait()

        scatter_row(0, 0)
        start_dma(0, 0)
        scatter_row(1, 1)
        start_dma(1, 1)

        @pl.loop(2, RPW, step=2)
        def _(r0):
            for p in range(2):
                r = r0 + p
                wait_dma(r - 2, p)
                unscatter_row(r - 2, p)
                scatter_row(r, p)
                start_dma(r, p)

        wait_dma(RPW - 2, 0)
        wait_dma(RPW - 1, 1)

    return body(idx, w)


def _tc_matmul(x_bf, s, bias_row):
    def body(x_ref, s_ref, b_ref, o_ref):
        sb = s_ref[...].astype(jnp.bfloat16)
        acc = lax.dot_general(x_ref[...], sb, (((1,), (1,)), ((), ())),
                              preferred_element_type=jnp.float32)
        o_ref[...] = acc + b_ref[...]

    return pl.pallas_call(
        body,
        grid=(OUT_FEATURES // TO,),
        in_specs=[
            pl.BlockSpec((B, IN_FEATURES), lambda j: (0, 0)),
            pl.BlockSpec((TO, IN_FEATURES), lambda j: (j, 0)),
            pl.BlockSpec((1, TO), lambda j: (0, j)),
        ],
        out_specs=pl.BlockSpec((B, TO), lambda j: (0, j)),
        out_shape=jax.ShapeDtypeStruct((B, OUT_FEATURES), jnp.float32),
    )(x_bf, s, bias_row)


def kernel(x, indices, weight, bias):
    idx = indices.astype(jnp.int32)
    s = _sc_scatter_weights(idx, weight.astype(jnp.float32))
    x_bf = x.astype(jnp.bfloat16)
    return _tc_matmul(x_bf, s, bias.astype(jnp.float32).reshape(1, OUT_FEATURES))
